# Initial kernel scaffold; baseline (speedup 1.0000x reference)
import jax
import jax.numpy as jnp
from jax.experimental import pallas as pl


def kernel(x, w0, b0, g0, be0, rm0, rv0, w1, b1, g1, be1, rm1, rv1, w2, b2, g2, be2, rm2, rv2, w3, b3, g3, be3, rm3, rv3, w4, b4, g4, be4, rm4, rv4, w5, b5, g5, be5, rm5, rv5):
    raise NotImplementedError("write your pallas kernel here")


# speedup vs baseline: 66.5034x; 66.5034x over previous
import functools

import jax
import jax.numpy as jnp
import numpy as np
from jax.experimental import pallas as pl
from jax.experimental.pallas import tpu as pltpu

_LAYER_CONFIGS = [
    (1, 10, 1),
    (10, 32, 2),
    (32, 64, 2),
    (64, 128, 2),
    (128, 256, 2),
    (256, 256, 2),
]
_BN_EPS = 1e-5

_BATCH_BLOCK = 1024


def _tap_index_map(h, w, stride):
    ho = (h + 2 - 3) // stride + 1
    wo = (w + 2 - 3) // stride + 1
    idx = np.full((ho * wo, h * w), 9, np.int32)
    for kh in range(3):
        for kw in range(3):
            t = kh * 3 + kw
            for i in range(ho):
                for j in range(wo):
                    r = stride * i + kh - 1
                    c = stride * j + kw - 1
                    if 0 <= r < h and 0 <= c < w:
                        idx[i * wo + j, r * w + c] = t
    return idx, ho, wo


def _dense_layer(w, b, g, be, rm, rv, idx):
    cout, cin = w.shape[0], w.shape[1]
    m_out, m_in = idx.shape
    scale = g / jnp.sqrt(rv + _BN_EPS)
    w_taps = jnp.transpose(w, (2, 3, 1, 0)).reshape(9, cin, cout)
    w_taps = w_taps * scale[None, None, :]
    w_pad = jnp.concatenate([w_taps, jnp.zeros((1, cin, cout), w_taps.dtype)], axis=0)
    mat = w_pad[idx]
    mat = jnp.transpose(mat, (1, 2, 0, 3)).reshape(m_in * cin, m_out * cout)
    bias = (b - rm) * scale + be
    bias_row = jnp.tile(bias, m_out).reshape(1, m_out * cout)
    return mat.astype(jnp.bfloat16), bias_row.astype(jnp.float32)


def _mlp_kernel(x_ref, *refs):
    o_ref = refs[-1]
    h = x_ref[...]
    for li in range(5):
        m_ref, b_ref = refs[2 * li], refs[2 * li + 1]
        acc = jnp.dot(h, m_ref[...], preferred_element_type=jnp.float32)
        h = jnp.maximum(acc + b_ref[...], 0.0).astype(jnp.bfloat16)
    acc = jnp.dot(h, refs[10][...], preferred_element_type=jnp.float32)
    o_ref[...] = jnp.maximum(acc + refs[11][...], 0.0)


def kernel(x, w0, b0, g0, be0, rm0, rv0, w1, b1, g1, be1, rm1, rv1,
           w2, b2, g2, be2, rm2, rv2, w3, b3, g3, be3, rm3, rv3,
           w4, b4, g4, be4, rm4, rv4, w5, b5, g5, be5, rm5, rv5):
    params = [
        (w0, b0, g0, be0, rm0, rv0),
        (w1, b1, g1, be1, rm1, rv1),
        (w2, b2, g2, be2, rm2, rv2),
        (w3, b3, g3, be3, rm3, rv3),
        (w4, b4, g4, be4, rm4, rv4),
        (w5, b5, g5, be5, rm5, rv5),
    ]
    n, cin0, h, w = x.shape
    assert cin0 == 1

    consts = []
    cur_h, cur_w = h, w
    for li in range(6):
        idx, ho, wo = _tap_index_map(cur_h, cur_w, _LAYER_CONFIGS[li][2])
        mat, bias_row = _dense_layer(*params[li], idx)
        consts += [mat, bias_row]
        cur_h, cur_w = ho, wo
    cout_last = _LAYER_CONFIGS[-1][1]

    x_flat = x.reshape(n, h * w).astype(jnp.bfloat16)

    nb = _BATCH_BLOCK if n % _BATCH_BLOCK == 0 else 8
    in_specs = [pl.BlockSpec((nb, h * w), lambda i: (i, 0))]
    in_specs += [pl.BlockSpec(c.shape, lambda i: (0, 0)) for c in consts]

    out = pl.pallas_call(
        _mlp_kernel,
        out_shape=jax.ShapeDtypeStruct((n, cout_last), jnp.float32),
        grid=(n // nb,),
        in_specs=in_specs,
        out_specs=pl.BlockSpec((nb, cout_last), lambda i: (i, 0)),
        compiler_params=pltpu.CompilerParams(
            dimension_semantics=("parallel",),
            vmem_limit_bytes=100 * 1024 * 1024,
        ),
    )(x_flat, *consts)
    return out.reshape(n, cout_last, 1, 1)


# speedup vs baseline: 97.0690x; 1.4596x over previous
import functools

import jax
import jax.numpy as jnp
import numpy as np
from jax.experimental import pallas as pl
from jax.experimental.pallas import tpu as pltpu

_LAYER_CONFIGS = [
    (1, 10, 1),
    (10, 32, 2),
    (32, 64, 2),
    (64, 128, 2),
    (128, 256, 2),
    (256, 256, 2),
]
_BN_EPS = 1e-5

_BATCH_BLOCK = 1024


def _tap_select(h, w, stride):
    ho = (h + 2 - 3) // stride + 1
    wo = (w + 2 - 3) // stride + 1
    a = np.zeros((9, h * w, ho * wo), np.float32)
    for kh in range(3):
        for kw in range(3):
            t = kh * 3 + kw
            for i in range(ho):
                for j in range(wo):
                    r = stride * i + kh - 1
                    c = stride * j + kw - 1
                    if 0 <= r < h and 0 <= c < w:
                        a[t, r * w + c, i * wo + j] = 1.0
    return a, ho, wo


def _dense_layer(w, b, g, be, rm, rv, a_sel):
    cout, cin = w.shape[0], w.shape[1]
    _, m_in, m_out = a_sel.shape
    scale = g / jnp.sqrt(rv + _BN_EPS)
    w_taps = jnp.transpose(w, (2, 3, 1, 0)).reshape(9, cin, cout)
    w_taps = w_taps * scale[None, None, :]
    mat = (w_taps[:, :, None, :, None] * a_sel[:, None, :, None, :]).sum(0)
    mat = mat.reshape(cin * m_in, cout * m_out)
    bias = (b - rm) * scale + be
    bias_row = jnp.repeat(bias, m_out).reshape(1, cout * m_out)
    return mat.astype(jnp.bfloat16), bias_row.astype(jnp.float32)


def _mlp_kernel(x_ref, *refs):
    o_ref = refs[-1]
    h = x_ref[...].astype(jnp.bfloat16)
    for li in range(5):
        m_ref, b_ref = refs[2 * li], refs[2 * li + 1]
        acc = jnp.dot(h, m_ref[...], preferred_element_type=jnp.float32)
        h = jnp.maximum(acc + b_ref[...], 0.0).astype(jnp.bfloat16)
    acc = jnp.dot(h, refs[10][...], preferred_element_type=jnp.float32)
    o_ref[...] = jnp.maximum(acc + refs[11][...], 0.0)


def kernel(x, w0, b0, g0, be0, rm0, rv0, w1, b1, g1, be1, rm1, rv1,
           w2, b2, g2, be2, rm2, rv2, w3, b3, g3, be3, rm3, rv3,
           w4, b4, g4, be4, rm4, rv4, w5, b5, g5, be5, rm5, rv5):
    params = [
        (w0, b0, g0, be0, rm0, rv0),
        (w1, b1, g1, be1, rm1, rv1),
        (w2, b2, g2, be2, rm2, rv2),
        (w3, b3, g3, be3, rm3, rv3),
        (w4, b4, g4, be4, rm4, rv4),
        (w5, b5, g5, be5, rm5, rv5),
    ]
    n, cin0, h, w = x.shape
    assert cin0 == 1

    consts = []
    cur_h, cur_w = h, w
    for li in range(6):
        a_sel, ho, wo = _tap_select(cur_h, cur_w, _LAYER_CONFIGS[li][2])
        mat, bias_row = _dense_layer(*params[li], a_sel)
        consts += [mat, bias_row]
        cur_h, cur_w = ho, wo
    cout_last = _LAYER_CONFIGS[-1][1]

    x_flat = x.reshape(n, h * w)

    nb = _BATCH_BLOCK if n % _BATCH_BLOCK == 0 else 8
    in_specs = [pl.BlockSpec((nb, h * w), lambda i: (i, 0))]
    in_specs += [pl.BlockSpec(c.shape, lambda i: (0, 0)) for c in consts]

    out = pl.pallas_call(
        _mlp_kernel,
        out_shape=jax.ShapeDtypeStruct((n, cout_last), jnp.float32),
        grid=(n // nb,),
        in_specs=in_specs,
        out_specs=pl.BlockSpec((nb, cout_last), lambda i: (i, 0)),
        compiler_params=pltpu.CompilerParams(
            dimension_semantics=("parallel",),
            vmem_limit_bytes=100 * 1024 * 1024,
        ),
    )(x_flat, *consts)
    return out.reshape(n, cout_last, 1, 1)


# speedup vs baseline: 159.0841x; 1.6389x over previous
import functools

import jax
import jax.numpy as jnp
import numpy as np
from jax.experimental import pallas as pl
from jax.experimental.pallas import tpu as pltpu

_LAYER_CONFIGS = [
    (1, 10, 1),
    (10, 32, 2),
    (32, 64, 2),
    (64, 128, 2),
    (128, 256, 2),
    (256, 256, 2),
]
_BN_EPS = 1e-5

_BLOCK = 256
_BATCH_BLOCK = 1024

_SPATIAL = [(16, 16), (16, 16), (8, 8), (4, 4), (2, 2), (1, 1)]


def _fold_bn(w, b, g, be, rm, rv):
    scale = g / jnp.sqrt(rv + _BN_EPS)
    w_taps = jnp.transpose(w, (2, 3, 1, 0)).reshape(9, w.shape[1], w.shape[0])
    return w_taps * scale[None, None, :], (b - rm) * scale + be


def _layer1_matrix(w_taps):
    a = np.zeros((9, 256, 16, 16), np.float32)
    for kh in range(3):
        for kw in range(3):
            t = kh * 3 + kw
            for r in range(16):
                for j in range(16):
                    ri, ci = r + kh - 1, j + kw - 1
                    if 0 <= ri < 16 and 0 <= ci < 16:
                        a[t, ri * 16 + ci, r, j] = 1.0
    a = jnp.asarray(a)
    wt = w_taps[:, 0, :]
    m = (wt[:, None, None, :, None] * a[:, :, :, None, :]).sum(0)
    m = m.reshape(256, 16, 160)
    m = jnp.pad(m, ((0, 0), (0, 0), (0, _BLOCK - 160)))
    return m.reshape(256, 16 * _BLOCK).astype(jnp.bfloat16)


def _band_matrix(w_taps, stride, w_in, w_out, khs):
    cin, cout = w_taps.shape[1], w_taps.shape[2]
    nr = len(khs)
    a = np.zeros((9, nr, w_in, w_out), np.float32)
    for rl, kh in enumerate(khs):
        for kw in range(3):
            t = kh * 3 + kw
            for jo in range(w_out):
                ji = stride * jo + kw - 1
                if 0 <= ji < w_in:
                    a[t, rl, ji, jo] = 1.0
    a = jnp.asarray(a)
    m = (w_taps[:, None, :, None, :, None] * a[:, :, None, :, None, :]).sum(0)
    m = m.reshape(nr, cin * w_in, cout * w_out)
    m = jnp.pad(m, ((0, 0), (0, _BLOCK - cin * w_in), (0, 0)))
    return m.reshape(nr * _BLOCK, cout * w_out).astype(jnp.bfloat16)


def _bias_block(bias, w_out):
    row = jnp.repeat(bias, w_out)
    row = jnp.pad(row, (0, _BLOCK - row.shape[0]))
    return row.reshape(1, _BLOCK).astype(jnp.float32)


def _bands(h_in, stride, h_out):
    out = []
    for k in range(h_out):
        rows = [r for r in (stride * k - 1, stride * k, stride * k + 1)
                if 0 <= r < h_in and abs(r - stride * k) <= 1]
        khs = [r - (stride * k - 1) for r in rows]
        out.append((khs, rows[0]))
    return out


def _net_kernel(x_ref, *refs, plan):
    o_ref = refs[-1]
    m1, b1 = refs[0], refs[1]
    x = x_ref[...].astype(jnp.bfloat16)
    acc = jnp.dot(x, m1[...], preferred_element_type=jnp.float32)
    h = jnp.maximum(acc + b1[...], 0.0).astype(jnp.bfloat16)

    ri = 2
    for li, bands in enumerate(plan):
        b_ref = refs[ri + len(bands)]
        outs = []
        for (nr, r0) in bands:
            seg = h[:, r0 * _BLOCK:(r0 + nr) * _BLOCK]
            acc = jnp.dot(seg, refs[ri][...], preferred_element_type=jnp.float32)
            y = jnp.maximum(acc + b_ref[...], 0.0)
            if li < len(plan) - 1:
                y = y.astype(jnp.bfloat16)
            outs.append(y)
            ri += 1
        ri += 1
        h = outs[0] if len(outs) == 1 else jnp.concatenate(outs, axis=1)
    o_ref[...] = h


def kernel(x, w0, b0, g0, be0, rm0, rv0, w1, b1, g1, be1, rm1, rv1,
           w2, b2, g2, be2, rm2, rv2, w3, b3, g3, be3, rm3, rv3,
           w4, b4, g4, be4, rm4, rv4, w5, b5, g5, be5, rm5, rv5):
    params = [
        (w0, b0, g0, be0, rm0, rv0),
        (w1, b1, g1, be1, rm1, rv1),
        (w2, b2, g2, be2, rm2, rv2),
        (w3, b3, g3, be3, rm3, rv3),
        (w4, b4, g4, be4, rm4, rv4),
        (w5, b5, g5, be5, rm5, rv5),
    ]
    n, cin0, h0, w0_ = x.shape
    assert cin0 == 1 and (h0, w0_) == (16, 16)

    wt1, bias1 = _fold_bn(*params[0])
    b1_row = jnp.pad(jnp.repeat(bias1, 16), (0, _BLOCK - 160))
    consts = [_layer1_matrix(wt1),
              jnp.tile(b1_row, 16).reshape(1, 16 * _BLOCK).astype(jnp.float32)]

    plan = []
    for li in range(1, 6):
        _, _, stride = _LAYER_CONFIGS[li]
        h_in, w_in = _SPATIAL[li]
        h_out, w_out = (h_in + 1) // stride, (w_in + 1) // stride
        wt, bias = _fold_bn(*params[li])
        bands = _bands(h_in, stride, h_out)
        for khs, r0 in bands:
            consts.append(_band_matrix(wt, stride, w_in, w_out, khs))
        consts.append(_bias_block(bias, w_out))
        plan.append(tuple((len(khs), r0) for khs, r0 in bands))

    x_flat = x.reshape(n, 256)

    nb = _BATCH_BLOCK if n % _BATCH_BLOCK == 0 else 8
    in_specs = [pl.BlockSpec((nb, 256), lambda i: (i, 0))]
    in_specs += [pl.BlockSpec(c.shape, lambda i: (0,) * c.ndim) for c in consts]

    out = pl.pallas_call(
        functools.partial(_net_kernel, plan=tuple(plan)),
        out_shape=jax.ShapeDtypeStruct((n, _BLOCK), jnp.float32),
        grid=(n // nb,),
        in_specs=in_specs,
        out_specs=pl.BlockSpec((nb, _BLOCK), lambda i: (i, 0)),
        compiler_params=pltpu.CompilerParams(
            dimension_semantics=("parallel",),
            vmem_limit_bytes=100 * 1024 * 1024,
        ),
    )(x_flat, *consts)
    return out.reshape(n, _BLOCK, 1, 1)


# speedup vs baseline: 160.0332x; 1.0060x over previous
import functools

import jax
import jax.numpy as jnp
import numpy as np
from jax.experimental import pallas as pl
from jax.experimental.pallas import tpu as pltpu

_LAYER_CONFIGS = [
    (1, 10, 1),
    (10, 32, 2),
    (32, 64, 2),
    (64, 128, 2),
    (128, 256, 2),
    (256, 256, 2),
]
_BN_EPS = 1e-5

_BLOCK = 256
_BATCH_BLOCK = 1024

_SPATIAL = [(16, 16), (16, 16), (8, 8), (4, 4), (2, 2), (1, 1)]


def _fold_bn(w, b, g, be, rm, rv):
    scale = g / jnp.sqrt(rv + _BN_EPS)
    w_taps = jnp.transpose(w, (2, 3, 1, 0)).reshape(9, w.shape[1], w.shape[0])
    return w_taps * scale[None, None, :], (b - rm) * scale + be


def _layer1_matrix(w_taps):
    a = np.zeros((9, 256, 16, 16), np.float32)
    for kh in range(3):
        for kw in range(3):
            t = kh * 3 + kw
            for r in range(16):
                for j in range(16):
                    ri, ci = r + kh - 1, j + kw - 1
                    if 0 <= ri < 16 and 0 <= ci < 16:
                        a[t, ri * 16 + ci, r, j] = 1.0
    a = jnp.asarray(a)
    wt = w_taps[:, 0, :]
    m = (wt[:, None, None, :, None] * a[:, :, :, None, :]).sum(0)
    m = m.reshape(256, 16, 160)
    m = jnp.pad(m, ((0, 0), (0, 0), (0, _BLOCK - 160)))
    return m.reshape(256, 16 * _BLOCK).astype(jnp.bfloat16)


def _band_matrix(w_taps, stride, w_in, w_out, khs):
    cin, cout = w_taps.shape[1], w_taps.shape[2]
    nr = len(khs)
    a = np.zeros((9, nr, w_in, w_out), np.float32)
    for rl, kh in enumerate(khs):
        for kw in range(3):
            t = kh * 3 + kw
            for jo in range(w_out):
                ji = stride * jo + kw - 1
                if 0 <= ji < w_in:
                    a[t, rl, ji, jo] = 1.0
    a = jnp.asarray(a)
    m = (w_taps[:, None, :, None, :, None] * a[:, :, None, :, None, :]).sum(0)
    m = m.reshape(nr, cin * w_in, cout * w_out)
    m = jnp.pad(m, ((0, 0), (0, _BLOCK - cin * w_in), (0, 0)))
    return m.reshape(nr * _BLOCK, cout * w_out).astype(jnp.bfloat16)


def _bias_block(bias, w_out):
    row = jnp.repeat(bias, w_out)
    row = jnp.pad(row, (0, _BLOCK - row.shape[0]))
    return row.reshape(1, _BLOCK).astype(jnp.float32)


def _bands(h_in, stride, h_out):
    out = []
    for k in range(h_out):
        rows = [r for r in (stride * k - 1, stride * k, stride * k + 1)
                if 0 <= r < h_in]
        khs = tuple(r - (stride * k - 1) for r in rows)
        out.append((khs, rows[0]))
    return out


def _net_kernel(x_ref, *refs, plan):
    o_ref = refs[-1]
    m1, b1 = refs[0], refs[1]
    x = x_ref[...].astype(jnp.bfloat16)
    acc = jnp.dot(x, m1[...], preferred_element_type=jnp.float32)
    h = jnp.maximum(acc + b1[...], 0.0).astype(jnp.bfloat16)

    ri = 2
    for li, (n_mats, bands) in enumerate(plan):
        mat_refs = refs[ri:ri + n_mats]
        b_ref = refs[ri + n_mats]
        outs = []
        for (nr, r0, mi) in bands:
            seg = h[:, r0 * _BLOCK:(r0 + nr) * _BLOCK]
            acc = jnp.dot(seg, mat_refs[mi][...], preferred_element_type=jnp.float32)
            y = jnp.maximum(acc + b_ref[...], 0.0)
            if li < len(plan) - 1:
                y = y.astype(jnp.bfloat16)
            outs.append(y)
        ri += n_mats + 1
        h = outs[0] if len(outs) == 1 else jnp.concatenate(outs, axis=1)
    o_ref[...] = h


def kernel(x, w0, b0, g0, be0, rm0, rv0, w1, b1, g1, be1, rm1, rv1,
           w2, b2, g2, be2, rm2, rv2, w3, b3, g3, be3, rm3, rv3,
           w4, b4, g4, be4, rm4, rv4, w5, b5, g5, be5, rm5, rv5):
    params = [
        (w0, b0, g0, be0, rm0, rv0),
        (w1, b1, g1, be1, rm1, rv1),
        (w2, b2, g2, be2, rm2, rv2),
        (w3, b3, g3, be3, rm3, rv3),
        (w4, b4, g4, be4, rm4, rv4),
        (w5, b5, g5, be5, rm5, rv5),
    ]
    n, cin0, h0, w0_ = x.shape
    assert cin0 == 1 and (h0, w0_) == (16, 16)

    wt1, bias1 = _fold_bn(*params[0])
    b1_row = jnp.pad(jnp.repeat(bias1, 16), (0, _BLOCK - 160))
    consts = [_layer1_matrix(wt1),
              jnp.tile(b1_row, 16).reshape(1, 16 * _BLOCK).astype(jnp.float32)]

    plan = []
    for li in range(1, 6):
        _, _, stride = _LAYER_CONFIGS[li]
        h_in, w_in = _SPATIAL[li]
        h_out, w_out = (h_in + 1) // stride, (w_in + 1) // stride
        wt, bias = _fold_bn(*params[li])
        mat_slot = {}
        layer_bands = []
        for khs, r0 in _bands(h_in, stride, h_out):
            if khs not in mat_slot:
                mat_slot[khs] = len(mat_slot)
                consts.append(_band_matrix(wt, stride, w_in, w_out, khs))
            layer_bands.append((len(khs), r0, mat_slot[khs]))
        consts.append(_bias_block(bias, w_out))
        plan.append((len(mat_slot), tuple(layer_bands)))

    x_flat = x.reshape(n, 256)

    nb = _BATCH_BLOCK if n % _BATCH_BLOCK == 0 else 8
    in_specs = [pl.BlockSpec((nb, 256), lambda i: (i, 0))]
    in_specs += [pl.BlockSpec(c.shape, lambda i: (0,) * c.ndim) for c in consts]

    out = pl.pallas_call(
        functools.partial(_net_kernel, plan=tuple(plan)),
        out_shape=jax.ShapeDtypeStruct((n, _BLOCK), jnp.float32),
        grid=(n // nb,),
        in_specs=in_specs,
        out_specs=pl.BlockSpec((nb, _BLOCK), lambda i: (i, 0)),
        compiler_params=pltpu.CompilerParams(
            dimension_semantics=("parallel",),
            vmem_limit_bytes=100 * 1024 * 1024,
        ),
    )(x_flat, *consts)
    return out.reshape(n, _BLOCK, 1, 1)
